# Initial kernel scaffold; baseline (speedup 1.0000x reference)
import jax
import jax.numpy as jnp
from jax.experimental import pallas as pl


def kernel(x, edge_index, edge_type, edge_type_num, emb_W, emb_b, basis, comp, root_W, rgcn_b, rel_emb):
    raise NotImplementedError("write your pallas kernel here")


# speedup vs baseline: 17.2840x; 17.2840x over previous
import jax
import jax.numpy as jnp
from jax import lax
from jax.experimental import pallas as pl
from jax.experimental.pallas import tpu as pltpu
from jax.experimental.pallas import tpu_sc as plsc

N = 10000
E = 160000
EMB_DIM = 10
OUT_DIM = 6
NUM_ET = 4

HW = 16
ONE_COL = EMB_DIM
NR = 10240
ACC_ROWS = NUM_ET * NR
NC = 2
NS = 16
NWORK = NC * NS
JCH = 40
CHW = 128
EPT = JCH * CHW
E_PAD = NWORK * EPT
RPT = ACC_ROWS // NS
H2W = 8


def _encode_body(x_ref, w_ref, b_ref, out_ref):
    out_ref[...] = (
        jnp.dot(x_ref[...], w_ref[...], preferred_element_type=jnp.float32)
        + b_ref[...]
    )


def _encode(x_pad, emb_W_pad, emb_b_pad):
    return pl.pallas_call(
        _encode_body,
        out_shape=jax.ShapeDtypeStruct((NR, HW), jnp.float32),
    )(x_pad, emb_W_pad, emb_b_pad)


def _scatter_body(h_hbm, src_hbm, sidx_hbm, zeros_hbm, acc_out,
                  s2, idx2, rows, acc_sh, sem):
    cid = lax.axis_index("c")
    sid = lax.axis_index("s")
    wid = sid * NC + cid

    pltpu.sync_copy(zeros_hbm.at[pl.ds(sid * RPT, RPT)],
                    rows.at[pl.ds(0, RPT)])
    pltpu.sync_copy(rows.at[pl.ds(0, RPT)],
                    acc_sh.at[pl.ds(sid * RPT, RPT)])
    plsc.subcore_barrier()

    pltpu.sync_copy(src_hbm.at[wid], s2)
    pltpu.sync_copy(sidx_hbm.at[wid], idx2)

    def jloop(j, carry):
        pltpu.async_copy(h_hbm.at[s2.at[j]], rows.at[pl.ds(0, CHW)], sem).wait()
        pltpu.sync_copy(rows.at[pl.ds(0, CHW)], acc_sh.at[idx2.at[j]],
                        add=True)
        return carry

    lax.fori_loop(0, JCH, jloop, 0)
    plsc.subcore_barrier()

    pltpu.sync_copy(acc_sh.at[pl.ds(sid * RPT, RPT)], rows.at[pl.ds(0, RPT)])
    pltpu.sync_copy(rows.at[pl.ds(0, RPT)],
                    acc_out.at[cid, pl.ds(sid * RPT, RPT)])


def _scatter(h_pad, src3, sidx3, zeros):
    mesh = plsc.VectorSubcoreMesh(core_axis_name="c", subcore_axis_name="s")
    fn = pl.kernel(
        _scatter_body,
        out_type=jax.ShapeDtypeStruct((NC, ACC_ROWS, HW), jnp.float32),
        mesh=mesh,
        compiler_params=pltpu.CompilerParams(use_tc_tiling_on_sc=False),
        scratch_types=[
            pltpu.VMEM((JCH, CHW), jnp.int32),
            pltpu.VMEM((JCH, CHW), jnp.int32),
            pltpu.VMEM((RPT, HW), jnp.float32),
            pltpu.VMEM_SHARED((ACC_ROWS, HW), jnp.float32),
            pltpu.SemaphoreType.DMA,
        ],
    )
    return fn(h_pad, src3, sidx3, zeros)


BR = 2560


def _combine_body(acc_ref, h_ref, comp_ref, basis_ref, rootW_ref, b_ref,
                  out_ref):
    S = acc_ref[0] + acc_ref[1]
    cnt = jnp.maximum(S[:, :, ONE_COL], 1.0)
    comp = comp_ref[...]
    msg = jnp.zeros((BR, OUT_DIM), jnp.float32)
    for r in range(NUM_ET):
        mean_r = S[r, :, :EMB_DIM] / cnt[r][:, None]
        relW_r = comp[r, 0] * basis_ref[0] + comp[r, 1] * basis_ref[1]
        msg = msg + jnp.dot(mean_r, relW_r, preferred_element_type=jnp.float32)
    root = jnp.dot(h_ref[:, :EMB_DIM], rootW_ref[...],
                   preferred_element_type=jnp.float32) + b_ref[...]
    h2 = jnp.maximum(root + msg, 0.0)
    out_ref[...] = jnp.pad(h2, ((0, 0), (0, H2W - OUT_DIM)))


def _combine(acc, h_pad, comp, basis, root_W, rgcn_b2):
    return pl.pallas_call(
        _combine_body,
        grid=(NR // BR,),
        in_specs=[
            pl.BlockSpec((NC, NUM_ET, BR, HW), lambda i: (0, 0, i, 0)),
            pl.BlockSpec((BR, HW), lambda i: (i, 0)),
            pl.BlockSpec((NUM_ET, 2), lambda i: (0, 0)),
            pl.BlockSpec((2, EMB_DIM, OUT_DIM), lambda i: (0, 0, 0)),
            pl.BlockSpec((EMB_DIM, OUT_DIM), lambda i: (0, 0)),
            pl.BlockSpec((1, OUT_DIM), lambda i: (0, 0)),
        ],
        out_specs=pl.BlockSpec((BR, H2W), lambda i: (i, 0)),
        out_shape=jax.ShapeDtypeStruct((NR, H2W), jnp.float32),
    )(acc, h_pad, comp, basis, root_W, rgcn_b2)


def _decode_body(h2_hbm, rel_hbm, src_hbm, dst_hbm, typ_hbm, out_hbm,
                 h2v, relv, s1, d1, t1, scv):
    cid = lax.axis_index("c")
    sid = lax.axis_index("s")
    wid = sid * NC + cid

    pltpu.sync_copy(h2_hbm, h2v)
    pltpu.sync_copy(rel_hbm, relv)
    pltpu.sync_copy(src_hbm.at[wid], s1)
    pltpu.sync_copy(dst_hbm.at[wid], d1)
    pltpu.sync_copy(typ_hbm.at[wid], t1)

    def iloop(i, carry):
        sl = pl.ds(i * 16, 16)
        s16 = s1[sl]
        d16 = d1[sl]
        t16 = t1[sl]
        acc = jnp.zeros((16,), jnp.float32)
        for k in range(OUT_DIM):
            kf = jnp.full((16,), k, jnp.int32)
            hs = plsc.load_gather(h2v, [s16, kf])
            hd = plsc.load_gather(h2v, [d16, kf])
            dk = plsc.load_gather(relv, [t16, kf])
            acc = acc + hs * hd * dk
        scv[sl] = 1.0 / (1.0 + jnp.exp(-acc))
        return carry

    lax.fori_loop(0, EPT // 16, iloop, 0)
    pltpu.sync_copy(scv, out_hbm.at[wid])


def _decode(h2p, rel_pad, src2, dst2, typ2):
    mesh = plsc.VectorSubcoreMesh(core_axis_name="c", subcore_axis_name="s")
    fn = pl.kernel(
        _decode_body,
        out_type=jax.ShapeDtypeStruct((NWORK, EPT), jnp.float32),
        mesh=mesh,
        compiler_params=pltpu.CompilerParams(
            use_tc_tiling_on_sc=False, needs_layout_passes=False),
        scratch_types=[
            pltpu.VMEM((NR, H2W), jnp.float32),
            pltpu.VMEM((NUM_ET, H2W), jnp.float32),
            pltpu.VMEM((EPT,), jnp.int32),
            pltpu.VMEM((EPT,), jnp.int32),
            pltpu.VMEM((EPT,), jnp.int32),
            pltpu.VMEM((EPT,), jnp.float32),
        ],
    )
    return fn(h2p, rel_pad, src2, dst2, typ2)


def kernel(x, edge_index, edge_type, edge_type_num, emb_W, emb_b,
           basis, comp, root_W, rgcn_b, rel_emb):
    del edge_type_num

    emb_W_pad = jnp.pad(emb_W, ((0, 0), (0, HW - EMB_DIM)))
    emb_b_pad = jnp.pad(emb_b, (0, HW - EMB_DIM)).at[ONE_COL].set(1.0)
    emb_b_pad = emb_b_pad[None, :]
    rgcn_b2 = rgcn_b[None, :]
    rel_pad = jnp.pad(rel_emb, ((0, 0), (0, H2W - OUT_DIM)))

    pad = E_PAD - E
    src = jnp.concatenate([edge_index[0], jnp.zeros((pad,), jnp.int32)])
    dst = jnp.concatenate([edge_index[1], jnp.full((pad,), NR - 1, jnp.int32)])
    typ = jnp.concatenate([edge_type, jnp.full((pad,), NUM_ET - 1, jnp.int32)])
    sidx = typ * NR + dst
    src3 = src.reshape(NWORK, JCH, CHW)
    sidx3 = sidx.reshape(NWORK, JCH, CHW)
    src2 = src.reshape(NWORK, EPT)
    dst2 = dst.reshape(NWORK, EPT)
    typ2 = typ.reshape(NWORK, EPT)
    zeros = jnp.zeros((ACC_ROWS, HW), jnp.float32)
    x_pad = jnp.pad(x, ((0, NR - N), (0, 0)))

    h_pad = _encode(x_pad, emb_W_pad, emb_b_pad)
    acc = _scatter(h_pad, src3, sidx3, zeros)
    acc4 = acc.reshape(NC, NUM_ET, NR, HW)
    h2p = _combine(acc4, h_pad, comp, basis, root_W, rgcn_b2)
    scores2 = _decode(h2p, rel_pad, src2, dst2, typ2)
    return scores2.reshape(E_PAD)[:E]


# speedup vs baseline: 20.5670x; 1.1899x over previous
import jax
import jax.numpy as jnp
from jax import lax
from jax.experimental import pallas as pl
from jax.experimental.pallas import tpu as pltpu
from jax.experimental.pallas import tpu_sc as plsc

N = 10000
E = 160000
EMB_DIM = 10
OUT_DIM = 6
NUM_ET = 4

HW = 16
ONE_COL = EMB_DIM
NR = 10240
ACC_ROWS = NUM_ET * NR
NC = 2
NS = 16
NWORK = NC * NS
JCH = 40
CHW = 125
EPT = JCH * CHW
RPT = ACC_ROWS // NS
TPR = NS // NUM_ET
NCH = (EPT + 15) // 16


def _encode_body(x_ref, w_ref, b_ref, out_ref):
    out_ref[...] = (
        jnp.dot(x_ref[...], w_ref[...], preferred_element_type=jnp.float32)
        + b_ref[...]
    )


def _encode(x, emb_W_pad, emb_b_pad):
    return pl.pallas_call(
        _encode_body,
        out_shape=jax.ShapeDtypeStruct((N, HW), jnp.float32),
    )(x, emb_W_pad, emb_b_pad)


def _scatter_body(h_hbm, src_hbm, sidx_hbm, zeros_hbm, acc_out,
                  s2, idx2, rows, bufa, bufb, acc_sh, sema, semb):
    cid = lax.axis_index("c")
    sid = lax.axis_index("s")
    wid = sid * NC + cid

    pltpu.sync_copy(zeros_hbm, rows)
    pltpu.sync_copy(rows, acc_sh.at[pl.ds(sid * RPT, RPT)])
    plsc.subcore_barrier()

    pltpu.sync_copy(src_hbm.at[wid], s2)
    pltpu.sync_copy(sidx_hbm.at[wid], idx2)

    bufs = (bufa, bufb)
    sems = (sema, semb)
    pltpu.async_copy(h_hbm.at[s2.at[0]], bufs[0], sems[0])

    def jloop(jj, carry):
        for b in range(2):
            j = 2 * jj + b
            nxt = j + 1

            @pl.when(nxt < JCH)
            def _():
                pltpu.async_copy(h_hbm.at[s2.at[nxt]], bufs[1 - b],
                                 sems[1 - b])

            pltpu.make_async_copy(h_hbm.at[s2.at[j]], bufs[b],
                                  sems[b]).wait()
            pltpu.sync_copy(bufs[b], acc_sh.at[idx2.at[j]], add=True)
        return carry

    lax.fori_loop(0, JCH // 2, jloop, 0)
    plsc.subcore_barrier()

    r = sid // TPR
    local = (sid % TPR) * RPT
    pltpu.sync_copy(acc_sh.at[pl.ds(sid * RPT, RPT)], rows)
    pltpu.sync_copy(rows, acc_out.at[cid, r, pl.ds(local, RPT)])


def _scatter(h_pad, src3, sidx3, zeros):
    mesh = plsc.VectorSubcoreMesh(core_axis_name="c", subcore_axis_name="s")
    fn = pl.kernel(
        _scatter_body,
        out_type=jax.ShapeDtypeStruct((NC, NUM_ET, NR, HW), jnp.float32),
        mesh=mesh,
        compiler_params=pltpu.CompilerParams(use_tc_tiling_on_sc=False),
        scratch_types=[
            pltpu.VMEM((JCH, CHW), jnp.int32),
            pltpu.VMEM((JCH, CHW), jnp.int32),
            pltpu.VMEM((RPT, HW), jnp.float32),
            pltpu.VMEM((CHW, HW), jnp.float32),
            pltpu.VMEM((CHW, HW), jnp.float32),
            pltpu.VMEM_SHARED((ACC_ROWS, HW), jnp.float32),
            pltpu.SemaphoreType.DMA,
            pltpu.SemaphoreType.DMA,
        ],
    )
    return fn(h_pad, src3, sidx3, zeros)


BR = 2000


def _combine_body(acc_ref, h_ref, comp_ref, basis_ref, rootW_ref, b_ref,
                  out_ref):
    S = acc_ref[0] + acc_ref[1]
    cnt = jnp.maximum(S[:, :, ONE_COL], 1.0)
    comp = comp_ref[...]
    msg = jnp.zeros((BR, OUT_DIM), jnp.float32)
    for r in range(NUM_ET):
        mean_r = S[r, :, :EMB_DIM] / cnt[r][:, None]
        relW_r = comp[r, 0] * basis_ref[0] + comp[r, 1] * basis_ref[1]
        msg = msg + jnp.dot(mean_r, relW_r, preferred_element_type=jnp.float32)
    root = jnp.dot(h_ref[:, :EMB_DIM], rootW_ref[...],
                   preferred_element_type=jnp.float32) + b_ref[...]
    out_ref[...] = jnp.maximum(root + msg, 0.0)


def _combine(acc, h_pad, comp, basis, root_W, rgcn_b2):
    return pl.pallas_call(
        _combine_body,
        grid=(N // BR,),
        in_specs=[
            pl.BlockSpec((NC, NUM_ET, BR, HW), lambda i: (0, 0, i, 0)),
            pl.BlockSpec((BR, HW), lambda i: (i, 0)),
            pl.BlockSpec((NUM_ET, 2), lambda i: (0, 0)),
            pl.BlockSpec((2, EMB_DIM, OUT_DIM), lambda i: (0, 0, 0)),
            pl.BlockSpec((EMB_DIM, OUT_DIM), lambda i: (0, 0)),
            pl.BlockSpec((1, OUT_DIM), lambda i: (0, 0)),
        ],
        out_specs=pl.BlockSpec((BR, OUT_DIM), lambda i: (i, 0)),
        out_shape=jax.ShapeDtypeStruct((N, OUT_DIM), jnp.float32),
    )(acc, h_pad, comp, basis, root_W, rgcn_b2)


def _decode_body(h2_hbm, rel_hbm, src_hbm, dst_hbm, typ_hbm, out_hbm,
                 h2v, relv, s1, d1, t1, scv):
    cid = lax.axis_index("c")
    sid = lax.axis_index("s")
    wid = sid * NC + cid

    pltpu.sync_copy(h2_hbm, h2v)
    pltpu.sync_copy(rel_hbm, relv)
    pltpu.sync_copy(src_hbm.at[wid], s1)
    pltpu.sync_copy(dst_hbm.at[wid], d1)
    pltpu.sync_copy(typ_hbm.at[wid], t1)

    def iloop(i, carry):
        start = jnp.minimum(i * 16, EPT - 16)
        sl = pl.ds(start, 16)
        s16 = s1[sl]
        d16 = d1[sl]
        t16 = t1[sl]
        acc = jnp.zeros((16,), jnp.float32)
        for k in range(OUT_DIM):
            kf = jnp.full((16,), k, jnp.int32)
            hs = plsc.load_gather(h2v, [s16, kf])
            hd = plsc.load_gather(h2v, [d16, kf])
            dk = plsc.load_gather(relv, [t16, kf])
            acc = acc + hs * hd * dk
        scv[sl] = 1.0 / (1.0 + jnp.exp(-acc))
        return carry

    lax.fori_loop(0, NCH, iloop, 0)
    pltpu.sync_copy(scv, out_hbm.at[wid])


def _decode(h2, rel_emb, src2, dst2, typ2):
    mesh = plsc.VectorSubcoreMesh(core_axis_name="c", subcore_axis_name="s")
    fn = pl.kernel(
        _decode_body,
        out_type=jax.ShapeDtypeStruct((NWORK, EPT), jnp.float32),
        mesh=mesh,
        compiler_params=pltpu.CompilerParams(
            use_tc_tiling_on_sc=False, needs_layout_passes=False),
        scratch_types=[
            pltpu.VMEM((N, OUT_DIM), jnp.float32),
            pltpu.VMEM((NUM_ET, OUT_DIM), jnp.float32),
            pltpu.VMEM((EPT,), jnp.int32),
            pltpu.VMEM((EPT,), jnp.int32),
            pltpu.VMEM((EPT,), jnp.int32),
            pltpu.VMEM((EPT,), jnp.float32),
        ],
    )
    return fn(h2, rel_emb, src2, dst2, typ2)


def kernel(x, edge_index, edge_type, edge_type_num, emb_W, emb_b,
           basis, comp, root_W, rgcn_b, rel_emb):
    del edge_type_num

    emb_W_pad = jnp.pad(emb_W, ((0, 0), (0, HW - EMB_DIM)))
    emb_b_pad = jnp.pad(emb_b, (0, HW - EMB_DIM)).at[ONE_COL].set(1.0)
    emb_b_pad = emb_b_pad[None, :]
    rgcn_b2 = rgcn_b[None, :]

    src = edge_index[0]
    dst = edge_index[1]
    sidx = edge_type * NR + dst
    src3 = src.reshape(NWORK, JCH, CHW)
    sidx3 = sidx.reshape(NWORK, JCH, CHW)
    src2 = src.reshape(NWORK, EPT)
    dst2 = dst.reshape(NWORK, EPT)
    typ2 = edge_type.reshape(NWORK, EPT)
    zeros = jnp.zeros((RPT, HW), jnp.float32)

    h_pad = _encode(x, emb_W_pad, emb_b_pad)
    acc = _scatter(h_pad, src3, sidx3, zeros)
    h2 = _combine(acc, h_pad, comp, basis, root_W, rgcn_b2)
    scores2 = _decode(h2, rel_emb, src2, dst2, typ2)
    return scores2.reshape(E)


# speedup vs baseline: 29.9286x; 1.4552x over previous
import jax
import jax.numpy as jnp
from jax import lax
from jax.experimental import pallas as pl
from jax.experimental.pallas import tpu as pltpu
from jax.experimental.pallas import tpu_sc as plsc

N = 10000
E = 160000
EMB_DIM = 10
OUT_DIM = 6
NUM_ET = 4

HW = 16
ONE_COL = EMB_DIM
NR = 10240
ACC_ROWS = NUM_ET * NR
NP = NR // 8
H2W = 8
NC = 2
NS = 16
NWORK = NC * NS
JCH = 40
CHW = 125
EPT = JCH * CHW
RPT = ACC_ROWS // NS
TPR = NS // NUM_ET
NCH = (EPT + 15) // 16


def _encode_body(x4_ref, w_ref, b_ref, out_ref):
    parts = [
        jnp.dot(x4_ref[:, j, :], w_ref[...], preferred_element_type=jnp.float32)
        for j in range(8)
    ]
    out_ref[...] = jnp.concatenate(parts, axis=1) + b_ref[...]


def _encode(x4, emb_W_pad, b_pack):
    return pl.pallas_call(
        _encode_body,
        out_shape=jax.ShapeDtypeStruct((NP, 128), jnp.float32),
    )(x4, emb_W_pad, b_pack)


def _scatter_body(h_hbm, src_hbm, sidx_hbm, zeros_hbm, acc_out,
                  s2, idx2, bufa, bufb, acc_sh, sema, semb):
    cid = lax.axis_index("c")
    sid = lax.axis_index("s")
    wid = sid * NC + cid

    pltpu.sync_copy(zeros_hbm, acc_sh.at[pl.ds(sid * RPT, RPT)])
    plsc.subcore_barrier()

    pltpu.sync_copy(src_hbm.at[wid], s2)
    pltpu.sync_copy(sidx_hbm.at[wid], idx2)

    bufs = (bufa, bufb)
    sems = (sema, semb)
    pltpu.async_copy(h_hbm.at[s2.at[0]], bufs[0], sems[0])

    def jloop(jj, carry):
        for b in range(2):
            j = 2 * jj + b
            nxt = j + 1

            @pl.when(nxt < JCH)
            def _():
                pltpu.async_copy(h_hbm.at[s2.at[nxt]], bufs[1 - b],
                                 sems[1 - b])

            pltpu.make_async_copy(h_hbm.at[s2.at[j]], bufs[b],
                                  sems[b]).wait()
            pltpu.sync_copy(bufs[b], acc_sh.at[idx2.at[j]], add=True)
        return carry

    lax.fori_loop(0, JCH // 2, jloop, 0)
    plsc.subcore_barrier()

    r = sid // TPR
    local = (sid % TPR) * RPT
    pltpu.sync_copy(acc_sh.at[pl.ds(sid * RPT, RPT)],
                    acc_out.at[cid, r, pl.ds(local, RPT)])


def _scatter(h_lin, src3, sidx3, zeros):
    mesh = plsc.VectorSubcoreMesh(core_axis_name="c", subcore_axis_name="s")
    fn = pl.kernel(
        _scatter_body,
        out_type=jax.ShapeDtypeStruct((NC, NUM_ET, NR, HW), jnp.float32),
        mesh=mesh,
        compiler_params=pltpu.CompilerParams(use_tc_tiling_on_sc=False),
        scratch_types=[
            pltpu.VMEM((JCH, CHW), jnp.int32),
            pltpu.VMEM((JCH, CHW), jnp.int32),
            pltpu.VMEM((CHW, HW), jnp.float32),
            pltpu.VMEM((CHW, HW), jnp.float32),
            pltpu.VMEM_SHARED((ACC_ROWS, HW), jnp.float32),
            pltpu.SemaphoreType.DMA,
            pltpu.SemaphoreType.DMA,
        ],
    )
    return fn(h_lin, src3, sidx3, zeros)


BPK = NP // 4


def _combine_body(acc_ref, h_ref, msel_ref, wbig_ref, rootbig_ref, bbig_ref,
                  out_ref):
    msg = jnp.zeros((BPK, 64), jnp.float32)
    for r in range(NUM_ET):
        Sr = acc_ref[0, r] + acc_ref[1, r]
        cntE = jnp.maximum(
            jnp.dot(Sr, msel_ref[...], preferred_element_type=jnp.float32),
            1.0)
        msg = msg + jnp.dot(Sr / cntE, wbig_ref[r],
                            preferred_element_type=jnp.float32)
    root = jnp.dot(h_ref[...], rootbig_ref[...],
                   preferred_element_type=jnp.float32) + bbig_ref[...]
    out_ref[...] = jnp.maximum(root + msg, 0.0)


def _combine(acc_packed, h_packed, msel, wbig, rootbig, bbig):
    return pl.pallas_call(
        _combine_body,
        grid=(NP // BPK,),
        in_specs=[
            pl.BlockSpec((NC, NUM_ET, BPK, 128), lambda i: (0, 0, i, 0)),
            pl.BlockSpec((BPK, 128), lambda i: (i, 0)),
            pl.BlockSpec((128, 128), lambda i: (0, 0)),
            pl.BlockSpec((NUM_ET, 128, 64), lambda i: (0, 0, 0)),
            pl.BlockSpec((128, 64), lambda i: (0, 0)),
            pl.BlockSpec((1, 64), lambda i: (0, 0)),
        ],
        out_specs=pl.BlockSpec((BPK, 64), lambda i: (i, 0)),
        out_shape=jax.ShapeDtypeStruct((NP, 64), jnp.float32),
    )(acc_packed, h_packed, msel, wbig, rootbig, bbig)


def _decode_body(h2_hbm, rel_hbm, src_hbm, dst_hbm, typ_hbm, out_hbm,
                 h2v, relv, s1, d1, t1, scv):
    cid = lax.axis_index("c")
    sid = lax.axis_index("s")
    wid = sid * NC + cid

    pltpu.sync_copy(h2_hbm, h2v)
    pltpu.sync_copy(rel_hbm, relv)
    pltpu.sync_copy(src_hbm.at[wid], s1)
    pltpu.sync_copy(dst_hbm.at[wid], d1)
    pltpu.sync_copy(typ_hbm.at[wid], t1)

    def iloop(i, carry):
        start = jnp.minimum(i * 16, EPT - 16)
        sl = pl.ds(start, 16)
        s16 = s1[sl]
        d16 = d1[sl]
        t16 = t1[sl]
        acc = jnp.zeros((16,), jnp.float32)
        for k in range(OUT_DIM):
            kf = jnp.full((16,), k, jnp.int32)
            hs = plsc.load_gather(h2v, [s16, kf])
            hd = plsc.load_gather(h2v, [d16, kf])
            dk = plsc.load_gather(relv, [t16, kf])
            acc = acc + hs * hd * dk
        scv[sl] = 1.0 / (1.0 + jnp.exp(-acc))
        return carry

    lax.fori_loop(0, NCH, iloop, 0)
    pltpu.sync_copy(scv, out_hbm.at[wid])


def _decode(h2_lin, rel_pad, src2, dst2, typ2):
    mesh = plsc.VectorSubcoreMesh(core_axis_name="c", subcore_axis_name="s")
    fn = pl.kernel(
        _decode_body,
        out_type=jax.ShapeDtypeStruct((NWORK, EPT), jnp.float32),
        mesh=mesh,
        compiler_params=pltpu.CompilerParams(
            use_tc_tiling_on_sc=False, needs_layout_passes=False),
        scratch_types=[
            pltpu.VMEM((NR, H2W), jnp.float32),
            pltpu.VMEM((NUM_ET, H2W), jnp.float32),
            pltpu.VMEM((EPT,), jnp.int32),
            pltpu.VMEM((EPT,), jnp.int32),
            pltpu.VMEM((EPT,), jnp.int32),
            pltpu.VMEM((EPT,), jnp.float32),
        ],
    )
    return fn(h2_lin, rel_pad, src2, dst2, typ2)


def kernel(x, edge_index, edge_type, edge_type_num, emb_W, emb_b,
           basis, comp, root_W, rgcn_b, rel_emb):
    del edge_type_num
    f32 = jnp.float32

    emb_W_pad = jnp.pad(emb_W, ((0, 0), (0, HW - EMB_DIM)))
    b_vec = jnp.pad(emb_b, (0, HW - EMB_DIM)).at[ONE_COL].set(1.0)
    b_pack = jnp.tile(b_vec, 8)[None, :]
    relW = jnp.einsum('rb,bio->rio', comp, basis)
    eye8 = jnp.eye(8, dtype=f32)
    relW_p = jnp.pad(relW, ((0, 0), (0, HW - EMB_DIM), (0, H2W - OUT_DIM)))
    wbig = jnp.stack([jnp.kron(eye8, relW_p[r]) for r in range(NUM_ET)])
    rootbig = jnp.kron(eye8, jnp.pad(root_W, ((0, HW - EMB_DIM),
                                              (0, H2W - OUT_DIM))))
    msel = jnp.kron(eye8, jnp.zeros((HW, HW), f32).at[ONE_COL, :].set(1.0))
    bbig = jnp.tile(jnp.pad(rgcn_b, (0, H2W - OUT_DIM)), 8)[None, :]
    rel_pad = jnp.pad(rel_emb, ((0, 0), (0, H2W - OUT_DIM)))

    src = edge_index[0]
    dst = edge_index[1]
    sidx = edge_type * NR + dst
    src3 = src.reshape(NWORK, JCH, CHW)
    sidx3 = sidx.reshape(NWORK, JCH, CHW)
    src2 = src.reshape(NWORK, EPT)
    dst2 = dst.reshape(NWORK, EPT)
    typ2 = edge_type.reshape(NWORK, EPT)
    zeros = jnp.zeros((RPT, HW), f32)
    x4 = jnp.pad(x, ((0, NR - N), (0, 0))).reshape(NP, 8, 128)

    h_packed = _encode(x4, emb_W_pad, b_pack)
    h_lin = h_packed.reshape(NR, HW)
    acc = _scatter(h_lin, src3, sidx3, zeros)
    acc_packed = acc.reshape(NC, NUM_ET, NP, 128)
    h2p = _combine(acc_packed, h_packed, msel, wbig, rootbig, bbig)
    h2_lin = h2p.reshape(NR, H2W)
    scores2 = _decode(h2_lin, rel_pad, src2, dst2, typ2)
    return scores2.reshape(E)


# speedup vs baseline: 31.9954x; 1.0691x over previous
import jax
import jax.numpy as jnp
from jax import lax
from jax.experimental import pallas as pl
from jax.experimental.pallas import tpu as pltpu
from jax.experimental.pallas import tpu_sc as plsc

N = 10000
E = 160000
EMB_DIM = 10
OUT_DIM = 6
NUM_ET = 4

HW = 16
ONE_COL = EMB_DIM
NR = 10240
ACC_ROWS = NUM_ET * NR
NP = NR // 8
H2W = 8
NC = 2
NS = 16
NWORK = NC * NS
JCH = 40
CHW = 125
EPT = JCH * CHW
RPT = ACC_ROWS // NS
TPR = NS // NUM_ET
NCH = (EPT + 15) // 16


def _encode_body(x4_ref, xt_ref, w_ref, b_ref, dst_ref, typ_ref,
                 out_ref, sidx_ref):
    x4 = jnp.concatenate([x4_ref[...], xt_ref[...]], axis=0)
    parts = [
        jnp.dot(x4[:, j, :], w_ref[...], preferred_element_type=jnp.float32)
        for j in range(8)
    ]
    out_ref[...] = jnp.concatenate(parts, axis=1) + b_ref[...]
    sidx_ref[...] = typ_ref[...] * NR + dst_ref[...]


def _encode(x4, xtail, emb_W_pad, b_pack, dstP, typP):
    return pl.pallas_call(
        _encode_body,
        out_shape=[
            jax.ShapeDtypeStruct((NP, 128), jnp.float32),
            jax.ShapeDtypeStruct((E // 128, 128), jnp.int32),
        ],
    )(x4, xtail, emb_W_pad, b_pack, dstP, typP)


NBUF = 4


def _scatter_body(h_hbm, src_hbm, sidx_hbm, zeros_hbm, acc_out,
                  s2, idx2, b0, b1, b2, b3, acc_sh,
                  g0, g1, g2, g3, s0s, s1s, s2s, s3s):
    cid = lax.axis_index("c")
    sid = lax.axis_index("s")
    wid = sid * NC + cid

    pltpu.sync_copy(zeros_hbm, acc_sh.at[pl.ds(sid * RPT, RPT)])
    plsc.subcore_barrier()

    pltpu.sync_copy(src_hbm.at[wid], s2)
    pltpu.sync_copy(sidx_hbm.at[wid], idx2)

    bufs = (b0, b1, b2, b3)
    gsem = (g0, g1, g2, g3)
    ssem = (s0s, s1s, s2s, s3s)
    for b in range(NBUF):
        pltpu.async_copy(h_hbm.at[s2.at[b]], bufs[b], gsem[b])

    def jloop(jj, carry):
        for b in range(NBUF):
            j = NBUF * jj + b
            pltpu.make_async_copy(h_hbm.at[s2.at[j]], bufs[b],
                                  gsem[b]).wait()
            pltpu.async_copy(bufs[b], acc_sh.at[idx2.at[j]], ssem[b],
                             add=True)
            nxt = j + NBUF

            @pl.when(nxt < JCH)
            def _():
                pltpu.make_async_copy(bufs[b], acc_sh.at[idx2.at[j]],
                                      ssem[b]).wait()
                pltpu.async_copy(h_hbm.at[s2.at[nxt]], bufs[b], gsem[b])
        return carry

    lax.fori_loop(0, JCH // NBUF, jloop, 0)
    for b in range(NBUF):
        pltpu.make_async_copy(bufs[b], acc_sh.at[idx2.at[JCH - NBUF + b]],
                              ssem[b]).wait()
    plsc.subcore_barrier()

    r = sid // TPR
    local = (sid % TPR) * RPT
    pltpu.sync_copy(acc_sh.at[pl.ds(sid * RPT, RPT)],
                    acc_out.at[cid, r, pl.ds(local, RPT)])


def _scatter(h_lin, src3, sidx3, zeros):
    mesh = plsc.VectorSubcoreMesh(core_axis_name="c", subcore_axis_name="s")
    fn = pl.kernel(
        _scatter_body,
        out_type=jax.ShapeDtypeStruct((NC, NUM_ET, NR, HW), jnp.float32),
        mesh=mesh,
        compiler_params=pltpu.CompilerParams(use_tc_tiling_on_sc=False),
        scratch_types=[
            pltpu.VMEM((JCH, CHW), jnp.int32),
            pltpu.VMEM((JCH, CHW), jnp.int32),
        ] + [pltpu.VMEM((CHW, HW), jnp.float32) for _ in range(NBUF)] + [
            pltpu.VMEM_SHARED((ACC_ROWS, HW), jnp.float32),
        ] + [pltpu.SemaphoreType.DMA for _ in range(2 * NBUF)],
    )
    return fn(h_lin, src3, sidx3, zeros)


BPK = NP // 4


def _combine_body(acc_ref, h_ref, msel_ref, wbig_ref, rootbig_ref, bbig_ref,
                  out_ref):
    msg = jnp.zeros((BPK, 64), jnp.float32)
    for r in range(NUM_ET):
        Sr = acc_ref[0, r] + acc_ref[1, r]
        cntE = jnp.maximum(
            jnp.dot(Sr, msel_ref[...], preferred_element_type=jnp.float32),
            1.0)
        msg = msg + jnp.dot(Sr / cntE, wbig_ref[r],
                            preferred_element_type=jnp.float32)
    root = jnp.dot(h_ref[...], rootbig_ref[...],
                   preferred_element_type=jnp.float32) + bbig_ref[...]
    out_ref[...] = jnp.maximum(root + msg, 0.0)


def _combine(acc_packed, h_packed, msel, wbig, rootbig, bbig):
    return pl.pallas_call(
        _combine_body,
        grid=(NP // BPK,),
        in_specs=[
            pl.BlockSpec((NC, NUM_ET, BPK, 128), lambda i: (0, 0, i, 0)),
            pl.BlockSpec((BPK, 128), lambda i: (i, 0)),
            pl.BlockSpec((128, 128), lambda i: (0, 0)),
            pl.BlockSpec((NUM_ET, 128, 64), lambda i: (0, 0, 0)),
            pl.BlockSpec((128, 64), lambda i: (0, 0)),
            pl.BlockSpec((1, 64), lambda i: (0, 0)),
        ],
        out_specs=pl.BlockSpec((BPK, 64), lambda i: (i, 0)),
        out_shape=jax.ShapeDtypeStruct((NP, 64), jnp.float32),
    )(acc_packed, h_packed, msel, wbig, rootbig, bbig)


def _vreg_pick(v, idx):
    dn = lax.GatherDimensionNumbers(
        offset_dims=(), collapsed_slice_dims=(0,), start_index_map=(0,))
    return lax.gather(v, idx[:, None], dn, (1,),
                      mode=lax.GatherScatterMode.PROMISE_IN_BOUNDS)


def _decode_body(h2_hbm, rel_hbm, src_hbm, dst_hbm, typ_hbm, out_hbm,
                 h2v, relv, s1, d1, t1, scv):
    cid = lax.axis_index("c")
    sid = lax.axis_index("s")
    wid = sid * NC + cid

    pltpu.sync_copy(h2_hbm, h2v)
    pltpu.sync_copy(rel_hbm, relv)
    pltpu.sync_copy(src_hbm.at[wid], s1)
    pltpu.sync_copy(dst_hbm.at[wid], d1)
    pltpu.sync_copy(typ_hbm.at[wid], t1)

    lane4 = jnp.bitwise_and(lax.iota(jnp.int32, 16), 3)
    relk = [plsc.load_gather(relv, [lane4 * H2W + k])
            for k in range(OUT_DIM)]

    def iloop(i, carry):
        start = jnp.minimum(i * 16, EPT - 16)
        sl = pl.ds(start, 16)
        s8 = s1[sl] * H2W
        d8 = d1[sl] * H2W
        t16 = t1[sl]
        acc = jnp.zeros((16,), jnp.float32)
        for k in range(OUT_DIM):
            hs = plsc.load_gather(h2v, [s8 + k])
            hd = plsc.load_gather(h2v, [d8 + k])
            acc = acc + hs * hd * _vreg_pick(relk[k], t16)
        scv[sl] = 1.0 / (1.0 + jnp.exp(-acc))
        return carry

    lax.fori_loop(0, NCH, iloop, 0)
    pltpu.sync_copy(scv, out_hbm.at[wid])


def _decode(h2_lin, rel_pad, src2, dst2, typ2):
    mesh = plsc.VectorSubcoreMesh(core_axis_name="c", subcore_axis_name="s")
    fn = pl.kernel(
        _decode_body,
        out_type=jax.ShapeDtypeStruct((NWORK, EPT), jnp.float32),
        mesh=mesh,
        compiler_params=pltpu.CompilerParams(
            use_tc_tiling_on_sc=False, needs_layout_passes=False),
        scratch_types=[
            pltpu.VMEM((NR * H2W,), jnp.float32),
            pltpu.VMEM((NUM_ET * H2W,), jnp.float32),
            pltpu.VMEM((EPT,), jnp.int32),
            pltpu.VMEM((EPT,), jnp.int32),
            pltpu.VMEM((EPT,), jnp.int32),
            pltpu.VMEM((EPT,), jnp.float32),
        ],
    )
    return fn(h2_lin, rel_pad, src2, dst2, typ2)


def kernel(x, edge_index, edge_type, edge_type_num, emb_W, emb_b,
           basis, comp, root_W, rgcn_b, rel_emb):
    del edge_type_num
    f32 = jnp.float32

    emb_W_pad = jnp.pad(emb_W, ((0, 0), (0, HW - EMB_DIM)))
    b_vec = jnp.pad(emb_b, (0, HW - EMB_DIM)).at[ONE_COL].set(1.0)
    b_pack = jnp.tile(b_vec, 8)[None, :]
    relW = jnp.einsum('rb,bio->rio', comp, basis)
    eye8 = jnp.eye(8, dtype=f32)
    relW_p = jnp.pad(relW, ((0, 0), (0, HW - EMB_DIM), (0, H2W - OUT_DIM)))
    wbig = jnp.stack([jnp.kron(eye8, relW_p[r]) for r in range(NUM_ET)])
    rootbig = jnp.kron(eye8, jnp.pad(root_W, ((0, HW - EMB_DIM),
                                              (0, H2W - OUT_DIM))))
    msel = jnp.kron(eye8, jnp.zeros((HW, HW), f32).at[ONE_COL, :].set(1.0))
    bbig = jnp.tile(jnp.pad(rgcn_b, (0, H2W - OUT_DIM)), 8)[None, :]
    rel_pad = jnp.pad(rel_emb, ((0, 0), (0, H2W - OUT_DIM)))

    src = edge_index[0]
    dst = edge_index[1]
    src3 = src.reshape(NWORK, JCH, CHW)
    src2 = src.reshape(NWORK, EPT)
    dst2 = dst.reshape(NWORK, EPT)
    typ2 = edge_type.reshape(NWORK, EPT)
    dstP = dst.reshape(E // 128, 128)
    typP = edge_type.reshape(E // 128, 128)
    zeros = jnp.zeros((RPT, HW), f32)
    x4 = x.reshape(N // 8, 8, 128)
    xtail = jnp.zeros(((NR - N) // 8, 8, 128), f32)

    h_packed, sidxP = _encode(x4, xtail, emb_W_pad, b_pack, dstP, typP)
    h_lin = h_packed.reshape(NR, HW)
    sidx3 = sidxP.reshape(NWORK, JCH, CHW)
    acc = _scatter(h_lin, src3, sidx3, zeros)
    acc_packed = acc.reshape(NC, NUM_ET, NP, 128)
    h2p = _combine(acc_packed, h_packed, msel, wbig, rootbig, bbig)
    h2_flat = h2p.reshape(NR * H2W)
    scores2 = _decode(h2_flat, rel_pad.reshape(NUM_ET * H2W),
                      src2, dst2, typ2)
    return scores2.reshape(E)


# speedup vs baseline: 33.8535x; 1.0581x over previous
import jax
import jax.numpy as jnp
from jax import lax
from jax.experimental import pallas as pl
from jax.experimental.pallas import tpu as pltpu
from jax.experimental.pallas import tpu_sc as plsc

N = 10000
E = 160000
EMB_DIM = 10
OUT_DIM = 6
NUM_ET = 4

HW = 16
ONE_COL = EMB_DIM
NR = 10240
ACC_ROWS = NUM_ET * NR
NP = NR // 8
H2W = 8
NC = 2
NS = 16
NWORK = NC * NS
JCH = 40
CHW = 125
EPT = JCH * CHW
RPT = ACC_ROWS // NS
TPR = NS // NUM_ET
NCH = (EPT + 15) // 16


def _encode_body(x4_ref, xt_ref, w_ref, b_ref, out_ref):
    x4 = jnp.concatenate([x4_ref[...], xt_ref[...]], axis=0)
    parts = [
        jnp.dot(x4[:, j, :], w_ref[...], preferred_element_type=jnp.float32)
        for j in range(8)
    ]
    out_ref[...] = jnp.concatenate(parts, axis=1) + b_ref[...]


def _encode(x4, xtail, emb_W_pad, b_pack):
    return pl.pallas_call(
        _encode_body,
        out_shape=jax.ShapeDtypeStruct((NP, 128), jnp.float32),
    )(x4, xtail, emb_W_pad, b_pack)


NBUF = 4


def _scatter_body(h_hbm, src_hbm, sidx_hbm, zeros_hbm, acc_out,
                  s2, idx2, b0, b1, b2, b3, acc_sh,
                  g0, g1, g2, g3, s0s, s1s, s2s, s3s):
    cid = lax.axis_index("c")
    sid = lax.axis_index("s")
    wid = sid * NC + cid

    pltpu.sync_copy(zeros_hbm, acc_sh.at[pl.ds(sid * RPT, RPT)])
    plsc.subcore_barrier()

    pltpu.sync_copy(src_hbm.at[wid], s2)
    pltpu.sync_copy(sidx_hbm.at[wid], idx2)

    bufs = (b0, b1, b2, b3)
    gsem = (g0, g1, g2, g3)
    ssem = (s0s, s1s, s2s, s3s)
    for b in range(NBUF):
        pltpu.async_copy(h_hbm.at[s2.at[b]], bufs[b], gsem[b])

    def jloop(jj, carry):
        for b in range(NBUF):
            j = NBUF * jj + b
            pltpu.make_async_copy(h_hbm.at[s2.at[j]], bufs[b],
                                  gsem[b]).wait()
            pltpu.async_copy(bufs[b], acc_sh.at[idx2.at[j]], ssem[b],
                             add=True)
            nxt = j + NBUF

            @pl.when(nxt < JCH)
            def _():
                pltpu.make_async_copy(bufs[b], acc_sh.at[idx2.at[j]],
                                      ssem[b]).wait()
                pltpu.async_copy(h_hbm.at[s2.at[nxt]], bufs[b], gsem[b])
        return carry

    lax.fori_loop(0, JCH // NBUF, jloop, 0)
    for b in range(NBUF):
        pltpu.make_async_copy(bufs[b], acc_sh.at[idx2.at[JCH - NBUF + b]],
                              ssem[b]).wait()
    plsc.subcore_barrier()

    r = sid // TPR
    local = (sid % TPR) * RPT
    pltpu.sync_copy(acc_sh.at[pl.ds(sid * RPT, RPT)],
                    acc_out.at[cid, r, pl.ds(local, RPT)])


def _scatter(h_lin, src3, sidx3, zeros):
    mesh = plsc.VectorSubcoreMesh(core_axis_name="c", subcore_axis_name="s")
    fn = pl.kernel(
        _scatter_body,
        out_type=jax.ShapeDtypeStruct((NC, NUM_ET, NR, HW), jnp.float32),
        mesh=mesh,
        compiler_params=pltpu.CompilerParams(use_tc_tiling_on_sc=False),
        scratch_types=[
            pltpu.VMEM((JCH, CHW), jnp.int32),
            pltpu.VMEM((JCH, CHW), jnp.int32),
        ] + [pltpu.VMEM((CHW, HW), jnp.float32) for _ in range(NBUF)] + [
            pltpu.VMEM_SHARED((ACC_ROWS, HW), jnp.float32),
        ] + [pltpu.SemaphoreType.DMA for _ in range(2 * NBUF)],
    )
    return fn(h_lin, src3, sidx3, zeros)


BPK = NP // 4


def _combine_body(acc_ref, h_ref, msel_ref, wbig_ref, rootbig_ref, bbig_ref,
                  out_ref):
    msg = jnp.zeros((BPK, 64), jnp.float32)
    for r in range(NUM_ET):
        Sr = acc_ref[0, r] + acc_ref[1, r]
        cntE = jnp.maximum(
            jnp.dot(Sr, msel_ref[...], preferred_element_type=jnp.float32),
            1.0)
        msg = msg + jnp.dot(Sr / cntE, wbig_ref[r],
                            preferred_element_type=jnp.float32)
    root = jnp.dot(h_ref[...], rootbig_ref[...],
                   preferred_element_type=jnp.float32) + bbig_ref[...]
    out_ref[...] = jnp.maximum(root + msg, 0.0)


def _combine(acc_packed, h_packed, msel, wbig, rootbig, bbig):
    return pl.pallas_call(
        _combine_body,
        grid=(NP // BPK,),
        in_specs=[
            pl.BlockSpec((NC, NUM_ET, BPK, 128), lambda i: (0, 0, i, 0)),
            pl.BlockSpec((BPK, 128), lambda i: (i, 0)),
            pl.BlockSpec((128, 128), lambda i: (0, 0)),
            pl.BlockSpec((NUM_ET, 128, 64), lambda i: (0, 0, 0)),
            pl.BlockSpec((128, 64), lambda i: (0, 0)),
            pl.BlockSpec((1, 64), lambda i: (0, 0)),
        ],
        out_specs=pl.BlockSpec((BPK, 64), lambda i: (i, 0)),
        out_shape=jax.ShapeDtypeStruct((NP, 64), jnp.float32),
    )(acc_packed, h_packed, msel, wbig, rootbig, bbig)


def _vreg_pick(v, idx):
    dn = lax.GatherDimensionNumbers(
        offset_dims=(), collapsed_slice_dims=(0,), start_index_map=(0,))
    return lax.gather(v, idx[:, None], dn, (1,),
                      mode=lax.GatherScatterMode.PROMISE_IN_BOUNDS)


def _decode_body(h2_hbm, rel_hbm, src_hbm, dst_hbm, typ_hbm, out_hbm,
                 h2v, relv, s1, d1, t1, scv):
    cid = lax.axis_index("c")
    sid = lax.axis_index("s")
    wid = sid * NC + cid

    pltpu.sync_copy(h2_hbm, h2v)
    pltpu.sync_copy(rel_hbm, relv)
    pltpu.sync_copy(src_hbm.at[wid], s1)
    pltpu.sync_copy(dst_hbm.at[wid], d1)
    pltpu.sync_copy(typ_hbm.at[wid], t1)

    lane4 = jnp.bitwise_and(lax.iota(jnp.int32, 16), 3)
    relk = [plsc.load_gather(relv, [lane4 * H2W + k])
            for k in range(OUT_DIM)]

    @plsc.parallel_loop(0, NCH, 1, unroll=4)
    def iloop(i):
        start = jnp.minimum(i * 16, EPT - 16)
        sl = pl.ds(start, 16)
        s8 = s1[sl] * H2W
        d8 = d1[sl] * H2W
        t16 = t1[sl]
        acc = jnp.zeros((16,), jnp.float32)
        for k in range(OUT_DIM):
            hs = plsc.load_gather(h2v, [s8 + k])
            hd = plsc.load_gather(h2v, [d8 + k])
            acc = acc + hs * hd * _vreg_pick(relk[k], t16)
        scv[sl] = 1.0 / (1.0 + jnp.exp(-acc))

    pltpu.sync_copy(scv, out_hbm.at[wid])


def _decode(h2_lin, rel_pad, src2, dst2, typ2):
    mesh = plsc.VectorSubcoreMesh(core_axis_name="c", subcore_axis_name="s")
    fn = pl.kernel(
        _decode_body,
        out_type=jax.ShapeDtypeStruct((NWORK, EPT), jnp.float32),
        mesh=mesh,
        compiler_params=pltpu.CompilerParams(
            use_tc_tiling_on_sc=False, needs_layout_passes=False),
        scratch_types=[
            pltpu.VMEM((NR * H2W,), jnp.float32),
            pltpu.VMEM((NUM_ET * H2W,), jnp.float32),
            pltpu.VMEM((EPT,), jnp.int32),
            pltpu.VMEM((EPT,), jnp.int32),
            pltpu.VMEM((EPT,), jnp.int32),
            pltpu.VMEM((EPT,), jnp.float32),
        ],
    )
    return fn(h2_lin, rel_pad, src2, dst2, typ2)


def kernel(x, edge_index, edge_type, edge_type_num, emb_W, emb_b,
           basis, comp, root_W, rgcn_b, rel_emb):
    del edge_type_num
    f32 = jnp.float32

    emb_W_pad = jnp.pad(emb_W, ((0, 0), (0, HW - EMB_DIM)))
    b_vec = jnp.pad(emb_b, (0, HW - EMB_DIM)).at[ONE_COL].set(1.0)
    b_pack = jnp.tile(b_vec, 8)[None, :]
    relW = jnp.einsum('rb,bio->rio', comp, basis)
    eye8 = jnp.eye(8, dtype=f32)
    relW_p = jnp.pad(relW, ((0, 0), (0, HW - EMB_DIM), (0, H2W - OUT_DIM)))
    wbig = jnp.stack([jnp.kron(eye8, relW_p[r]) for r in range(NUM_ET)])
    rootbig = jnp.kron(eye8, jnp.pad(root_W, ((0, HW - EMB_DIM),
                                              (0, H2W - OUT_DIM))))
    msel = jnp.kron(eye8, jnp.zeros((HW, HW), f32).at[ONE_COL, :].set(1.0))
    bbig = jnp.tile(jnp.pad(rgcn_b, (0, H2W - OUT_DIM)), 8)[None, :]
    rel_pad = jnp.pad(rel_emb, ((0, 0), (0, H2W - OUT_DIM)))

    src = edge_index[0]
    dst = edge_index[1]
    src3 = src.reshape(NWORK, JCH, CHW)
    src2 = src.reshape(NWORK, EPT)
    dst2 = dst.reshape(NWORK, EPT)
    typ2 = edge_type.reshape(NWORK, EPT)
    sidx3 = (edge_type * NR + dst).reshape(NWORK, JCH, CHW)
    zeros = jnp.zeros((RPT, HW), f32)
    x4 = x.reshape(N // 8, 8, 128)
    xtail = jnp.zeros(((NR - N) // 8, 8, 128), f32)

    h_packed = _encode(x4, xtail, emb_W_pad, b_pack)
    h_lin = h_packed.reshape(NR, HW)
    acc = _scatter(h_lin, src3, sidx3, zeros)
    acc_packed = acc.reshape(NC, NUM_ET, NP, 128)
    h2p = _combine(acc_packed, h_packed, msel, wbig, rootbig, bbig)
    h2_flat = h2p.reshape(NR * H2W)
    scores2 = _decode(h2_flat, rel_pad.reshape(NUM_ET * H2W),
                      src2, dst2, typ2)
    return scores2.reshape(E)


# speedup vs baseline: 34.6519x; 1.0236x over previous
import jax
import jax.numpy as jnp
from jax import lax
from jax.experimental import pallas as pl
from jax.experimental.pallas import tpu as pltpu
from jax.experimental.pallas import tpu_sc as plsc

N = 10000
E = 160000
EMB_DIM = 10
OUT_DIM = 6
NUM_ET = 4

HW = 16
ONE_COL = EMB_DIM
NR = 10240
ACC_ROWS = NUM_ET * NR
NP = NR // 8
H2W = 8
NC = 2
NS = 16
NWORK = NC * NS
JCH = 40
CHW = 125
EPT = JCH * CHW
RPT = ACC_ROWS // NS
TPR = NS // NUM_ET
NCH = (EPT + 15) // 16


def _encode_body(x4_ref, xt_ref, w_ref, b_ref, out_ref):
    x4 = jnp.concatenate([x4_ref[...], xt_ref[...]], axis=0)
    parts = [
        jnp.dot(x4[:, j, :], w_ref[...], preferred_element_type=jnp.float32)
        for j in range(8)
    ]
    out_ref[...] = jnp.concatenate(parts, axis=1) + b_ref[...]


def _encode(x4, xtail, emb_W_pad, b_pack):
    return pl.pallas_call(
        _encode_body,
        out_shape=jax.ShapeDtypeStruct((NP, 128), jnp.float32),
    )(x4, xtail, emb_W_pad, b_pack)


NBUF = 8


def _scatter_body(h_hbm, ei_hbm, sidx_hbm, zeros_hbm, acc_out,
                  s2, idx2, *rest):
    bufs = rest[:NBUF]
    acc_sh = rest[NBUF]
    gsem = rest[NBUF + 1:2 * NBUF + 1]
    ssem = rest[2 * NBUF + 1:]
    cid = lax.axis_index("c")
    sid = lax.axis_index("s")
    wid = sid * NC + cid

    pltpu.sync_copy(zeros_hbm, acc_sh.at[pl.ds(sid * RPT, RPT)])
    plsc.subcore_barrier()

    pltpu.sync_copy(ei_hbm.at[0, wid], s2)
    pltpu.sync_copy(sidx_hbm.at[wid], idx2)
    for b in range(NBUF):
        pltpu.async_copy(h_hbm.at[s2.at[b]], bufs[b], gsem[b])

    def jloop(jj, carry):
        for b in range(NBUF):
            j = NBUF * jj + b
            pltpu.make_async_copy(h_hbm.at[s2.at[j]], bufs[b],
                                  gsem[b]).wait()
            pltpu.async_copy(bufs[b], acc_sh.at[idx2.at[j]], ssem[b],
                             add=True)
            nxt = j + NBUF

            @pl.when(nxt < JCH)
            def _():
                pltpu.make_async_copy(bufs[b], acc_sh.at[idx2.at[j]],
                                      ssem[b]).wait()
                pltpu.async_copy(h_hbm.at[s2.at[nxt]], bufs[b], gsem[b])
        return carry

    lax.fori_loop(0, JCH // NBUF, jloop, 0)
    for b in range(NBUF):
        pltpu.make_async_copy(bufs[b], acc_sh.at[idx2.at[JCH - NBUF + b]],
                              ssem[b]).wait()
    plsc.subcore_barrier()

    r = sid // TPR
    local = (sid % TPR) * RPT
    pltpu.sync_copy(acc_sh.at[pl.ds(sid * RPT, RPT)],
                    acc_out.at[cid, r, pl.ds(local, RPT)])


def _scatter(h_lin, ei3, sidx3, zeros):
    mesh = plsc.VectorSubcoreMesh(core_axis_name="c", subcore_axis_name="s")
    fn = pl.kernel(
        _scatter_body,
        out_type=jax.ShapeDtypeStruct((NC, NUM_ET, NR, HW), jnp.float32),
        mesh=mesh,
        compiler_params=pltpu.CompilerParams(use_tc_tiling_on_sc=False),
        scratch_types=[
            pltpu.VMEM((JCH, CHW), jnp.int32),
            pltpu.VMEM((JCH, CHW), jnp.int32),
        ] + [pltpu.VMEM((CHW, HW), jnp.float32) for _ in range(NBUF)] + [
            pltpu.VMEM_SHARED((ACC_ROWS, HW), jnp.float32),
        ] + [pltpu.SemaphoreType.DMA for _ in range(2 * NBUF)],
    )
    return fn(h_lin, ei3, sidx3, zeros)


BPK = NP // 4


def _combine_body(acc_ref, h_ref, msel_ref, wbig_ref, rootbig_ref, bbig_ref,
                  out_ref):
    msg = jnp.zeros((BPK, 64), jnp.float32)
    for r in range(NUM_ET):
        Sr = acc_ref[0, r] + acc_ref[1, r]
        cntE = jnp.maximum(
            jnp.dot(Sr, msel_ref[...], preferred_element_type=jnp.float32),
            1.0)
        msg = msg + jnp.dot(Sr / cntE, wbig_ref[r],
                            preferred_element_type=jnp.float32)
    root = jnp.dot(h_ref[...], rootbig_ref[...],
                   preferred_element_type=jnp.float32) + bbig_ref[...]
    out_ref[...] = jnp.maximum(root + msg, 0.0)


def _combine(acc_packed, h_packed, msel, wbig, rootbig, bbig):
    return pl.pallas_call(
        _combine_body,
        grid=(NP // BPK,),
        in_specs=[
            pl.BlockSpec((NC, NUM_ET, BPK, 128), lambda i: (0, 0, i, 0)),
            pl.BlockSpec((BPK, 128), lambda i: (i, 0)),
            pl.BlockSpec((128, 128), lambda i: (0, 0)),
            pl.BlockSpec((NUM_ET, 128, 64), lambda i: (0, 0, 0)),
            pl.BlockSpec((128, 64), lambda i: (0, 0)),
            pl.BlockSpec((1, 64), lambda i: (0, 0)),
        ],
        out_specs=pl.BlockSpec((BPK, 64), lambda i: (i, 0)),
        out_shape=jax.ShapeDtypeStruct((NP, 64), jnp.float32),
    )(acc_packed, h_packed, msel, wbig, rootbig, bbig)


def _vreg_pick(v, idx):
    dn = lax.GatherDimensionNumbers(
        offset_dims=(), collapsed_slice_dims=(0,), start_index_map=(0,))
    return lax.gather(v, idx[:, None], dn, (1,),
                      mode=lax.GatherScatterMode.PROMISE_IN_BOUNDS)


def _decode_body(h2_hbm, rel_hbm, ei_hbm, typ_hbm, out_hbm,
                 h2v, relv, s1, d1, t1, scv, sem):
    cid = lax.axis_index("c")
    sid = lax.axis_index("s")
    wid = sid * NC + cid

    pltpu.async_copy(h2_hbm, h2v, sem)
    pltpu.async_copy(rel_hbm, relv, sem)
    pltpu.async_copy(ei_hbm.at[0, wid], s1, sem)
    pltpu.async_copy(ei_hbm.at[1, wid], d1, sem)
    pltpu.async_copy(typ_hbm.at[wid], t1, sem)
    for ref, dst in ((h2_hbm, h2v), (rel_hbm, relv), (ei_hbm.at[0, wid], s1),
                     (ei_hbm.at[1, wid], d1), (typ_hbm.at[wid], t1)):
        pltpu.make_async_copy(ref, dst, sem).wait()

    lane4 = jnp.bitwise_and(lax.iota(jnp.int32, 16), 3)
    relk = [plsc.load_gather(relv, [lane4 * H2W + k])
            for k in range(OUT_DIM)]

    @plsc.parallel_loop(0, NCH, 1, unroll=8)
    def iloop(i):
        start = jnp.minimum(i * 16, EPT - 16)
        sl = pl.ds(start, 16)
        s8 = s1[sl] * H2W
        d8 = d1[sl] * H2W
        t16 = t1[sl]
        acc = jnp.zeros((16,), jnp.float32)
        for k in range(OUT_DIM):
            hs = plsc.load_gather(h2v, [s8 + k])
            hd = plsc.load_gather(h2v, [d8 + k])
            acc = acc + hs * hd * _vreg_pick(relk[k], t16)
        scv[sl] = 1.0 / (1.0 + jnp.exp(-acc))

    pltpu.sync_copy(scv, out_hbm.at[wid])


def _decode(h2_flat, rel_flat, ei2, typ2):
    mesh = plsc.VectorSubcoreMesh(core_axis_name="c", subcore_axis_name="s")
    fn = pl.kernel(
        _decode_body,
        out_type=jax.ShapeDtypeStruct((NWORK, EPT), jnp.float32),
        mesh=mesh,
        compiler_params=pltpu.CompilerParams(
            use_tc_tiling_on_sc=False, needs_layout_passes=False),
        scratch_types=[
            pltpu.VMEM((NR * H2W,), jnp.float32),
            pltpu.VMEM((NUM_ET * H2W,), jnp.float32),
            pltpu.VMEM((EPT,), jnp.int32),
            pltpu.VMEM((EPT,), jnp.int32),
            pltpu.VMEM((EPT,), jnp.int32),
            pltpu.VMEM((EPT,), jnp.float32),
            pltpu.SemaphoreType.DMA,
        ],
    )
    return fn(h2_flat, rel_flat, ei2, typ2)


def kernel(x, edge_index, edge_type, edge_type_num, emb_W, emb_b,
           basis, comp, root_W, rgcn_b, rel_emb):
    del edge_type_num
    f32 = jnp.float32

    emb_W_pad = jnp.pad(emb_W, ((0, 0), (0, HW - EMB_DIM)))
    b_vec = jnp.pad(emb_b, (0, HW - EMB_DIM)).at[ONE_COL].set(1.0)
    b_pack = jnp.tile(b_vec, 8)[None, :]
    relW = jnp.einsum('rb,bio->rio', comp, basis)
    eye8 = jnp.eye(8, dtype=f32)
    relW_p = jnp.pad(relW, ((0, 0), (0, HW - EMB_DIM), (0, H2W - OUT_DIM)))
    wbig = jnp.stack([jnp.kron(eye8, relW_p[r]) for r in range(NUM_ET)])
    rootbig = jnp.kron(eye8, jnp.pad(root_W, ((0, HW - EMB_DIM),
                                              (0, H2W - OUT_DIM))))
    msel = jnp.kron(eye8, jnp.zeros((HW, HW), f32).at[ONE_COL, :].set(1.0))
    bbig = jnp.tile(jnp.pad(rgcn_b, (0, H2W - OUT_DIM)), 8)[None, :]
    rel_pad = jnp.pad(rel_emb, ((0, 0), (0, H2W - OUT_DIM)))

    ei3 = edge_index.reshape(2, NWORK, JCH, CHW)
    ei2 = edge_index.reshape(2, NWORK, EPT)
    typ2 = edge_type.reshape(NWORK, EPT)
    sidx3 = (edge_type * NR + edge_index[1]).reshape(NWORK, JCH, CHW)
    zeros = jnp.zeros((RPT, HW), f32)
    x4 = x.reshape(N // 8, 8, 128)
    xtail = jnp.zeros(((NR - N) // 8, 8, 128), f32)

    h_packed = _encode(x4, xtail, emb_W_pad, b_pack)
    h_lin = h_packed.reshape(NR, HW)
    acc = _scatter(h_lin, ei3, sidx3, zeros)
    acc_packed = acc.reshape(NC, NUM_ET, NP, 128)
    h2p = _combine(acc_packed, h_packed, msel, wbig, rootbig, bbig)
    h2_flat = h2p.reshape(NR * H2W)
    scores2 = _decode(h2_flat, rel_pad.reshape(NUM_ET * H2W), ei2, typ2)
    return scores2.reshape(E)
